# baseline (speedup 1.0000x reference)
import functools

import jax
import jax.numpy as jnp
from jax import lax
from jax.experimental import pallas as pl
from jax.experimental.pallas import tpu as pltpu
from jax.experimental.pallas import tpu_sc as plsc

N_TOK = 102400
NSEG = 1024
HID = 256
NVOC = 100000

_NC, _NS, _L = 2, 16, 16
_NW = _NC * _NS
_CHUNK = N_TOK // _NW
_NVEC = _CHUNK // _L
_SEGW = NSEG // _NW
_KR = 128
_JC = _CHUNK // _KR
_ZR = NSEG // _NS

@functools.cache
def _mesh():
    return plsc.VectorSubcoreMesh(core_axis_name="c", subcore_axis_name="s",
                                  num_cores=_NC, num_subcores=_NS)


def _worker_id():
    return lax.axis_index("s") * _NC + lax.axis_index("c")


def _sc_boundaries(batch_ext):
    @functools.partial(
        pl.kernel,
        out_type=jax.ShapeDtypeStruct((_NW * NSEG,), jnp.int32),
        mesh=_mesh(),
        compiler_params=pltpu.CompilerParams(needs_layout_passes=False),
        scratch_types=[
            pltpu.VMEM((_CHUNK + _L,), jnp.int32),
            pltpu.VMEM((NSEG,), jnp.int32),
        ],
    )
    def k(batch_hbm, out_hbm, bvec, last_local):
        wid = _worker_id()
        base = wid * _CHUNK
        pltpu.sync_copy(batch_hbm.at[pl.ds(base, _CHUNK + _L)], bvec)
        zero = jnp.zeros((_L,), jnp.int32)

        def zbody(i, c):
            last_local[pl.ds(i * _L, _L)] = zero
            return c

        lax.fori_loop(0, NSEG // _L, zbody, 0)
        basev = lax.iota(jnp.int32, _L) + (base + 1)

        def body(j, c):
            cur = bvec[pl.ds(j * _L, _L)]
            nxt = bvec[pl.ds(j * _L + 1, _L)]
            vals = basev + j * _L
            plsc.store_scatter(last_local, [cur], vals, mask=cur != nxt)
            return c

        lax.fori_loop(0, _NVEC, body, 0)
        pltpu.sync_copy(last_local, out_hbm.at[pl.ds(wid * NSEG, NSEG)])

    return k(batch_ext)


def _sc_gather_vn(partials, emb):
    @functools.partial(
        pl.kernel,
        out_type=(jax.ShapeDtypeStruct((NSEG, HID), jnp.float32),
                  jax.ShapeDtypeStruct((NSEG,), jnp.int32)),
        mesh=_mesh(),
        compiler_params=pltpu.CompilerParams(needs_layout_passes=False),
        scratch_types=[
            pltpu.VMEM((_NW, _SEGW), jnp.int32),
            pltpu.VMEM((_SEGW,), jnp.int32),
            pltpu.VMEM((_SEGW,), jnp.int32),
            pltpu.VMEM((_SEGW, HID), jnp.float32),
            pltpu.SemaphoreType.DMA,
        ],
    )
    def k(part_hbm, emb_hbm, vn_hbm, lp_hbm, pm, idxv, lpv, rows, sem):
        wid = _worker_id()
        for j in range(_NW):
            pltpu.sync_copy(part_hbm.at[pl.ds(j * NSEG + wid * _SEGW, _SEGW)],
                            pm.at[j])
        acc0 = pm[0, pl.ds(0, _L)]
        acc1 = pm[0, pl.ds(_L, _L)]
        for j in range(1, _NW):
            acc0 = jnp.maximum(acc0, pm[j, pl.ds(0, _L)])
            acc1 = jnp.maximum(acc1, pm[j, pl.ds(_L, _L)])
        lpv[pl.ds(0, _L)] = acc0
        lpv[pl.ds(_L, _L)] = acc1
        idxv[pl.ds(0, _L)] = jnp.maximum(acc0 - 1, 0)
        idxv[pl.ds(_L, _L)] = jnp.maximum(acc1 - 1, 0)
        pltpu.async_copy(emb_hbm.at[idxv], rows, sem).wait()
        pltpu.sync_copy(rows, vn_hbm.at[pl.ds(wid * _SEGW, _SEGW)])
        pltpu.sync_copy(lpv, lp_hbm.at[pl.ds(wid * _SEGW, _SEGW)])

    return k(partials, emb)


_TRASH = _SEGW


def _sc_segment_sum(batch_ext, lp, ae_pad):
    @functools.partial(
        pl.kernel,
        out_type=jax.ShapeDtypeStruct((NSEG, HID), jnp.float32),
        mesh=_mesh(),
        compiler_params=pltpu.CompilerParams(needs_layout_passes=False),
        scratch_types=[
            pltpu.VMEM((NSEG,), jnp.int32),
            pltpu.VMEM((_KR,), jnp.int32),
            pltpu.VMEM((_KR, HID // 2), jnp.uint32),
            pltpu.VMEM((_KR, HID // 2), jnp.uint32),
            pltpu.VMEM((_SEGW + 8, HID), jnp.float32),
            pltpu.SemaphoreType.DMA,
            pltpu.SemaphoreType.DMA,
        ],
    )
    def k(bext_hbm, lp_hbm, ae_hbm, out_hbm, lpv, bv, rows0, rows1, acc, sem0, sem1):
        wid = _worker_id()
        sbase = wid * _SEGW
        pltpu.sync_copy(lp_hbm, lpv)
        zero = jnp.zeros((_L,), jnp.float32)

        def zrow(i, c):
            for q in range(HID // _L):
                acc[i, pl.ds(q * _L, _L)] = zero
            return c

        lax.fori_loop(0, _SEGW + 8, zrow, 0)

        def pbody(v, carry):
            m, s, e = carry
            s = jnp.where(v * _L == sbase, m, s)
            e = jnp.where(v * _L == sbase + _SEGW, m, e)
            m = jnp.maximum(m, jnp.max(lpv[pl.ds(v * _L, _L)]))
            return m, s, e

        m, start, end = lax.fori_loop(0, NSEG // _L, pbody,
                                      (jnp.int32(0), jnp.int32(0), jnp.int32(0)))
        end = jnp.where(sbase + _SEGW == NSEG, m, end)
        astart = (start // 8) * 8
        nch = (end - astart + _KR - 1) // _KR

        hmask = jnp.full((_L,), 0xFFFF0000, jnp.uint32)

        def cvt(u):
            fe = plsc.bitcast(u << 16, jnp.float32)
            fo = plsc.bitcast(u & hmask, jnp.float32)
            return fe, fo

        def accum(rows):
            def vbody(v, c):
                lvec = bv[pl.ds(v * _L, _L)] - sbase
                l0 = lvec[0]
                l15 = lvec[_L - 1]
                r0 = v * _L
                uni = (l0 == l15) & (l0 >= 0) & (l0 < _SEGW)

                @pl.when(uni)
                def _():
                    for j in range(HID // 32):
                        sl = pl.ds(j * _L, _L)
                        es = []
                        os_ = []
                        for t in range(_L):
                            fe, fo = cvt(rows[r0 + t, sl])
                            es.append(fe)
                            os_.append(fo)
                        while len(es) > 1:
                            es = [es[i] + es[i + 1] for i in range(0, len(es), 2)]
                            os_ = [os_[i] + os_[i + 1] for i in range(0, len(os_), 2)]
                        plsc.addupdate(acc.at[l0, pl.ds(j * _L, _L)], es[0])
                        plsc.addupdate(acc.at[l0, pl.ds(128 + j * _L, _L)], os_[0])

                @pl.when(jnp.logical_not(uni))
                def _():
                    for lane in range(_L):
                        lv = lvec[lane]
                        lv = jnp.where((lv >= 0) & (lv < _SEGW), lv, _TRASH)
                        for j in range(HID // 32):
                            fe, fo = cvt(rows[r0 + lane, pl.ds(j * _L, _L)])
                            plsc.addupdate(acc.at[lv, pl.ds(j * _L, _L)], fe)
                            plsc.addupdate(acc.at[lv, pl.ds(128 + j * _L, _L)], fo)
                return c

            lax.fori_loop(0, _KR // _L, vbody, 0)

        @pl.when(nch > 0)
        def _():
            pltpu.async_copy(ae_hbm.at[pl.ds(astart, _KR)], rows0, sem0)

        def body(i, carry):
            c0 = 2 * i
            c1 = c0 + 1
            c2 = c0 + 2
            off0 = astart + c0 * _KR

            @pl.when(c1 < nch)
            def _():
                pltpu.async_copy(ae_hbm.at[pl.ds(off0 + _KR, _KR)], rows1, sem1)

            pltpu.sync_copy(bext_hbm.at[pl.ds(off0, _KR)], bv)
            pltpu.make_async_copy(ae_hbm.at[pl.ds(off0, _KR)], rows0, sem0).wait()
            accum(rows0)

            @pl.when(c2 < nch)
            def _():
                pltpu.async_copy(ae_hbm.at[pl.ds(off0 + 2 * _KR, _KR)], rows0, sem0)

            @pl.when(c1 < nch)
            def _():
                pltpu.sync_copy(bext_hbm.at[pl.ds(off0 + _KR, _KR)], bv)
                pltpu.make_async_copy(ae_hbm.at[pl.ds(off0 + _KR, _KR)], rows1,
                                      sem1).wait()
                accum(rows1)

            return carry

        lax.fori_loop(0, (nch + 1) // 2, body, 0)
        pltpu.sync_copy(acc.at[pl.ds(0, _SEGW)], out_hbm.at[pl.ds(sbase, _SEGW)])

    return k(batch_ext, lp, ae_pad)


_TB = 1024


def _tc_alpha_ae(emb, batch2d, vn, W1, bias, W2, Wqp, bq2):
    def body(emb_ref, b_ref, vn_ref, w1_ref, bias_ref, w2_ref, wqp_ref, bq_ref,
             out_ref, t1_ref):
        @pl.when(pl.program_id(0) == 0)
        def _():
            t1_ref[...] = (
                lax.dot_general(vn_ref[...], w1_ref[...], (((1,), (1,)), ((), ())),
                                preferred_element_type=jnp.float32)
                + bias_ref[...]
            )

        t2 = lax.dot_general(emb_ref[...].astype(jnp.bfloat16),
                             w2_ref[...].astype(jnp.bfloat16),
                             (((1,), (1,)), ((), ())),
                             preferred_element_type=jnp.float32)
        ids = b_ref[0, 0].astype(jnp.int16)
        cols = lax.broadcasted_iota(jnp.int16, (_TB, NSEG), 1)
        oh = (ids.reshape(_TB, 1) == cols).astype(jnp.bfloat16)
        t1x = lax.dot_general(oh, t1_ref[...].astype(jnp.bfloat16),
                              (((1,), (0,)), ((), ())),
                              preferred_element_type=jnp.float32)
        pre = t2 + t1x
        sg = 1.0 / (1.0 + jnp.exp(-pre))
        af = lax.dot_general(sg.astype(jnp.bfloat16), wqp_ref[...],
                             (((1,), (0,)), ((), ())),
                             preferred_element_type=jnp.float32)
        alpha = af[:, 0:1] + bq_ref[...]
        out_ref[...] = (alpha * emb_ref[...]).astype(jnp.bfloat16)

    return pl.pallas_call(
        body,
        grid=(N_TOK // _TB,),
        in_specs=[
            pl.BlockSpec((_TB, HID), lambda i: (i, 0)),
            pl.BlockSpec((1, 1, _TB), lambda i: (i, 0, 0)),
            pl.BlockSpec((NSEG, HID), lambda i: (0, 0)),
            pl.BlockSpec((HID, HID), lambda i: (0, 0)),
            pl.BlockSpec((1, HID), lambda i: (0, 0)),
            pl.BlockSpec((HID, HID), lambda i: (0, 0)),
            pl.BlockSpec((HID, 128), lambda i: (0, 0)),
            pl.BlockSpec((1, 1), lambda i: (0, 0)),
        ],
        out_specs=pl.BlockSpec((_TB, HID), lambda i: (i, 0)),
        out_shape=jax.ShapeDtypeStruct((N_TOK + _KR, HID), jnp.bfloat16),
        scratch_shapes=[pltpu.VMEM((NSEG, HID), jnp.float32)],
    )(emb, batch2d, vn, W1, bias, W2, Wqp, bq2)


_VB = 2048
_NVB = (NVOC + _VB - 1) // _VB


def _tc_score(vn, sg, W3a, W3b, b3r, table):
    def body(vn_ref, sg_ref, w3a_ref, w3b_ref, b3_ref, tbl_ref, sh_ref, z_ref):
        @pl.when(pl.program_id(0) == 0)
        def _():
            sh_ref[...] = (
                lax.dot_general(vn_ref[...], w3a_ref[...], (((1,), (1,)), ((), ())),
                                preferred_element_type=jnp.float32)
                + lax.dot_general(sg_ref[...], w3b_ref[...], (((1,), (1,)), ((), ())),
                                  preferred_element_type=jnp.float32)
                + b3_ref[...]
            )

        z_ref[...] = lax.dot_general(sh_ref[...].astype(jnp.bfloat16),
                                     tbl_ref[...].astype(jnp.bfloat16),
                                     (((1,), (1,)), ((), ())),
                                     preferred_element_type=jnp.float32)

    return pl.pallas_call(
        body,
        grid=(_NVB,),
        in_specs=[
            pl.BlockSpec((NSEG, HID), lambda i: (0, 0)),
            pl.BlockSpec((NSEG, HID), lambda i: (0, 0)),
            pl.BlockSpec((HID, HID), lambda i: (0, 0)),
            pl.BlockSpec((HID, HID), lambda i: (0, 0)),
            pl.BlockSpec((1, HID), lambda i: (0, 0)),
            pl.BlockSpec((_VB, HID), lambda i: (i, 0)),
        ],
        out_specs=[
            pl.BlockSpec((NSEG, HID), lambda i: (0, 0)),
            pl.BlockSpec((NSEG, _VB), lambda i: (0, i)),
        ],
        out_shape=[
            jax.ShapeDtypeStruct((NSEG, HID), jnp.float32),
            jax.ShapeDtypeStruct((NSEG, NVOC), jnp.float32),
        ],
    )(vn, sg, W3a, W3b, b3r, table)


def kernel(session_embedding, batch, all_item_embedding, W1, b1, W2, b2, Wq, bq, W3, b3):
    batch = batch.astype(jnp.int32)
    batch_ext = jnp.concatenate([batch, jnp.full((_KR,), NSEG, jnp.int32)])

    batch2d = batch.reshape(N_TOK // _TB, 1, _TB)
    partials = _sc_boundaries(batch_ext)
    vn, lp = _sc_gather_vn(partials, session_embedding)
    wqp = jnp.pad(Wq.reshape(HID, 1).astype(jnp.bfloat16), ((0, 0), (0, 127)))
    ae_pad = _tc_alpha_ae(session_embedding, batch2d, vn, W1,
                          (b1 + b2).reshape(1, HID), W2, wqp, bq.reshape(1, 1))
    ae_u32 = jax.lax.bitcast_convert_type(
        ae_pad.reshape(N_TOK + _KR, HID // 2, 2), jnp.uint32)
    sg = _sc_segment_sum(batch_ext, lp, ae_u32)
    import numpy as _np
    _perm = _np.empty(HID, _np.int64)
    for _p in range(HID):
        _q, _r = (_p % 128) // 16, _p % 16
        _perm[_p] = 32 * _q + 2 * _r + (1 if _p >= 128 else 0)
    sh, z = _tc_score(vn, sg, W3[:, :HID], W3[:, HID:][:, _perm],
                      b3.reshape(1, HID), all_item_embedding)
    return sh, z


# speedup vs baseline: 1.6302x; 1.6302x over previous
import functools

import jax
import jax.numpy as jnp
from jax import lax
from jax.experimental import pallas as pl
from jax.experimental.pallas import tpu as pltpu
from jax.experimental.pallas import tpu_sc as plsc

N_TOK = 102400
NSEG = 1024
HID = 256
NVOC = 100000

_NC, _NS, _L = 2, 16, 16
_NW = _NC * _NS
_CHUNK = N_TOK // _NW
_NVEC = _CHUNK // _L
_SEGW = NSEG // _NW
_KR = 128
_JC = _CHUNK // _KR
_ZR = NSEG // _NS

@functools.cache
def _mesh():
    return plsc.VectorSubcoreMesh(core_axis_name="c", subcore_axis_name="s",
                                  num_cores=_NC, num_subcores=_NS)


def _worker_id():
    return lax.axis_index("s") * _NC + lax.axis_index("c")


def _sc_boundaries(batch_ext):
    @functools.partial(
        pl.kernel,
        out_type=jax.ShapeDtypeStruct((_NW * NSEG,), jnp.int32),
        mesh=_mesh(),
        compiler_params=pltpu.CompilerParams(needs_layout_passes=False),
        scratch_types=[
            pltpu.VMEM((_CHUNK + _L,), jnp.int32),
            pltpu.VMEM((NSEG,), jnp.int32),
        ],
    )
    def k(batch_hbm, out_hbm, bvec, last_local):
        wid = _worker_id()
        base = wid * _CHUNK
        pltpu.sync_copy(batch_hbm.at[pl.ds(base, _CHUNK + _L)], bvec)
        zero = jnp.zeros((_L,), jnp.int32)

        def zbody(i, c):
            last_local[pl.ds(i * _L, _L)] = zero
            return c

        lax.fori_loop(0, NSEG // _L, zbody, 0)
        basev = lax.iota(jnp.int32, _L) + (base + 1)

        def body(j, c):
            cur = bvec[pl.ds(j * _L, _L)]
            nxt = bvec[pl.ds(j * _L + 1, _L)]
            vals = basev + j * _L
            plsc.store_scatter(last_local, [cur], vals, mask=cur != nxt)
            return c

        lax.fori_loop(0, _NVEC, body, 0)
        pltpu.sync_copy(last_local, out_hbm.at[pl.ds(wid * NSEG, NSEG)])

    return k(batch_ext)


def _sc_gather_vn(partials, emb):
    @functools.partial(
        pl.kernel,
        out_type=(jax.ShapeDtypeStruct((NSEG, HID), jnp.float32),
                  jax.ShapeDtypeStruct((NSEG,), jnp.int32)),
        mesh=_mesh(),
        compiler_params=pltpu.CompilerParams(needs_layout_passes=False),
        scratch_types=[
            pltpu.VMEM((_NW, _SEGW), jnp.int32),
            pltpu.VMEM((_SEGW,), jnp.int32),
            pltpu.VMEM((_SEGW,), jnp.int32),
            pltpu.VMEM((_SEGW, HID), jnp.float32),
            pltpu.SemaphoreType.DMA,
        ],
    )
    def k(part_hbm, emb_hbm, vn_hbm, lp_hbm, pm, idxv, lpv, rows, sem):
        wid = _worker_id()
        for j in range(_NW):
            pltpu.sync_copy(part_hbm.at[pl.ds(j * NSEG + wid * _SEGW, _SEGW)],
                            pm.at[j])
        acc0 = pm[0, pl.ds(0, _L)]
        acc1 = pm[0, pl.ds(_L, _L)]
        for j in range(1, _NW):
            acc0 = jnp.maximum(acc0, pm[j, pl.ds(0, _L)])
            acc1 = jnp.maximum(acc1, pm[j, pl.ds(_L, _L)])
        lpv[pl.ds(0, _L)] = acc0
        lpv[pl.ds(_L, _L)] = acc1
        idxv[pl.ds(0, _L)] = jnp.maximum(acc0 - 1, 0)
        idxv[pl.ds(_L, _L)] = jnp.maximum(acc1 - 1, 0)
        pltpu.async_copy(emb_hbm.at[idxv], rows, sem).wait()
        pltpu.sync_copy(rows, vn_hbm.at[pl.ds(wid * _SEGW, _SEGW)])
        pltpu.sync_copy(lpv, lp_hbm.at[pl.ds(wid * _SEGW, _SEGW)])

    return k(partials, emb)


_TRASH = _SEGW


def _sc_segment_sum(batch_ext, lp, ae_pad):
    @functools.partial(
        pl.kernel,
        out_type=jax.ShapeDtypeStruct((NSEG, HID), jnp.float32),
        mesh=_mesh(),
        compiler_params=pltpu.CompilerParams(needs_layout_passes=False),
        scratch_types=[
            pltpu.VMEM((NSEG,), jnp.int32),
            pltpu.VMEM((_KR,), jnp.int32),
            pltpu.VMEM((_KR, HID // 2), jnp.uint32),
            pltpu.VMEM((_KR, HID // 2), jnp.uint32),
            pltpu.VMEM((_SEGW + 8, HID), jnp.float32),
            pltpu.SemaphoreType.DMA,
            pltpu.SemaphoreType.DMA,
        ],
    )
    def k(bext_hbm, lp_hbm, ae_hbm, out_hbm, lpv, bv, rows0, rows1, acc, sem0, sem1):
        wid = _worker_id()
        sbase = wid * _SEGW
        pltpu.sync_copy(lp_hbm, lpv)
        zero = jnp.zeros((_L,), jnp.float32)

        def zrow(i, c):
            for q in range(HID // _L):
                acc[i, pl.ds(q * _L, _L)] = zero
            return c

        lax.fori_loop(0, _SEGW + 8, zrow, 0)

        def pbody(v, carry):
            m, s, e = carry
            s = jnp.where(v * _L == sbase, m, s)
            e = jnp.where(v * _L == sbase + _SEGW, m, e)
            m = jnp.maximum(m, jnp.max(lpv[pl.ds(v * _L, _L)]))
            return m, s, e

        m, start, end = lax.fori_loop(0, NSEG // _L, pbody,
                                      (jnp.int32(0), jnp.int32(0), jnp.int32(0)))
        end = jnp.where(sbase + _SEGW == NSEG, m, end)
        astart = (start // 8) * 8
        nch = (end - astart + _KR - 1) // _KR

        hmask = jnp.full((_L,), 0xFFFF0000, jnp.uint32)

        def cvt(u):
            fe = plsc.bitcast(u << 16, jnp.float32)
            fo = plsc.bitcast(u & hmask, jnp.float32)
            return fe, fo

        def accum(rows):
            def vbody(v, c):
                lvec = bv[pl.ds(v * _L, _L)] - sbase
                l0 = lvec[0]
                l15 = lvec[_L - 1]
                r0 = v * _L
                uni = (l0 == l15) & (l0 >= 0) & (l0 < _SEGW)

                @pl.when(uni)
                def _():
                    for j in range(HID // 32):
                        sl = pl.ds(j * _L, _L)
                        es = []
                        os_ = []
                        for t in range(_L):
                            fe, fo = cvt(rows[r0 + t, sl])
                            es.append(fe)
                            os_.append(fo)
                        while len(es) > 1:
                            es = [es[i] + es[i + 1] for i in range(0, len(es), 2)]
                            os_ = [os_[i] + os_[i + 1] for i in range(0, len(os_), 2)]
                        plsc.addupdate(acc.at[l0, pl.ds(j * _L, _L)], es[0])
                        plsc.addupdate(acc.at[l0, pl.ds(128 + j * _L, _L)], os_[0])

                @pl.when(jnp.logical_not(uni))
                def _():
                    for lane in range(_L):
                        lv = lvec[lane]
                        lv = jnp.where((lv >= 0) & (lv < _SEGW), lv, _TRASH)
                        for j in range(HID // 32):
                            fe, fo = cvt(rows[r0 + lane, pl.ds(j * _L, _L)])
                            plsc.addupdate(acc.at[lv, pl.ds(j * _L, _L)], fe)
                            plsc.addupdate(acc.at[lv, pl.ds(128 + j * _L, _L)], fo)
                return c

            lax.fori_loop(0, _KR // _L, vbody, 0)

        @pl.when(nch > 0)
        def _():
            pltpu.async_copy(ae_hbm.at[pl.ds(astart, _KR)], rows0, sem0)

        def body(i, carry):
            c0 = 2 * i
            c1 = c0 + 1
            c2 = c0 + 2
            off0 = astart + c0 * _KR

            @pl.when(c1 < nch)
            def _():
                pltpu.async_copy(ae_hbm.at[pl.ds(off0 + _KR, _KR)], rows1, sem1)

            pltpu.sync_copy(bext_hbm.at[pl.ds(off0, _KR)], bv)
            pltpu.make_async_copy(ae_hbm.at[pl.ds(off0, _KR)], rows0, sem0).wait()
            accum(rows0)

            @pl.when(c2 < nch)
            def _():
                pltpu.async_copy(ae_hbm.at[pl.ds(off0 + 2 * _KR, _KR)], rows0, sem0)

            @pl.when(c1 < nch)
            def _():
                pltpu.sync_copy(bext_hbm.at[pl.ds(off0 + _KR, _KR)], bv)
                pltpu.make_async_copy(ae_hbm.at[pl.ds(off0 + _KR, _KR)], rows1,
                                      sem1).wait()
                accum(rows1)

            return carry

        lax.fori_loop(0, (nch + 1) // 2, body, 0)
        pltpu.sync_copy(acc.at[pl.ds(0, _SEGW)], out_hbm.at[pl.ds(sbase, _SEGW)])

    return k(batch_ext, lp, ae_pad)


_TB = 1024


def _tc_alpha_ae(emb, batch2d, vn, W1, bias, W2, Wqp, bq2):
    def body(emb_ref, b_ref, vn_ref, w1_ref, bias_ref, w2_ref, wqp_ref, bq_ref,
             out_ref, t1_ref):
        @pl.when(pl.program_id(0) == 0)
        def _():
            t1_ref[...] = (
                lax.dot_general(vn_ref[...], w1_ref[...], (((1,), (1,)), ((), ())),
                                preferred_element_type=jnp.float32)
                + bias_ref[...]
            )

        t2 = lax.dot_general(emb_ref[...].astype(jnp.bfloat16),
                             w2_ref[...].astype(jnp.bfloat16),
                             (((1,), (1,)), ((), ())),
                             preferred_element_type=jnp.float32)
        ids = b_ref[0, 0].astype(jnp.int16)
        cols = lax.broadcasted_iota(jnp.int16, (_TB, NSEG), 1)
        oh = (ids.reshape(_TB, 1) == cols).astype(jnp.bfloat16)
        t1x = lax.dot_general(oh, t1_ref[...].astype(jnp.bfloat16),
                              (((1,), (0,)), ((), ())),
                              preferred_element_type=jnp.float32)
        pre = t2 + t1x
        sg = 1.0 / (1.0 + jnp.exp(-pre))
        af = lax.dot_general(sg.astype(jnp.bfloat16), wqp_ref[...],
                             (((1,), (0,)), ((), ())),
                             preferred_element_type=jnp.float32)
        alpha = af[:, 0:1] + bq_ref[...]
        ab = alpha * emb_ref[...]
        lo = lax.bitcast_convert_type(ab[:, :128].astype(jnp.bfloat16),
                                      jnp.uint16).astype(jnp.uint32)
        hi = lax.bitcast_convert_type(ab[:, 128:].astype(jnp.bfloat16),
                                      jnp.uint16).astype(jnp.uint32)
        out_ref[...] = (hi << 16) | lo

    return pl.pallas_call(
        body,
        grid=(N_TOK // _TB,),
        in_specs=[
            pl.BlockSpec((_TB, HID), lambda i: (i, 0)),
            pl.BlockSpec((1, 1, _TB), lambda i: (i, 0, 0)),
            pl.BlockSpec((NSEG, HID), lambda i: (0, 0)),
            pl.BlockSpec((HID, HID), lambda i: (0, 0)),
            pl.BlockSpec((1, HID), lambda i: (0, 0)),
            pl.BlockSpec((HID, HID), lambda i: (0, 0)),
            pl.BlockSpec((HID, 128), lambda i: (0, 0)),
            pl.BlockSpec((1, 1), lambda i: (0, 0)),
        ],
        out_specs=pl.BlockSpec((_TB, HID // 2), lambda i: (i, 0)),
        out_shape=jax.ShapeDtypeStruct((N_TOK + _KR, HID // 2), jnp.uint32),
        scratch_shapes=[pltpu.VMEM((NSEG, HID), jnp.float32)],
    )(emb, batch2d, vn, W1, bias, W2, Wqp, bq2)


_VB = 2048
_NVB = (NVOC + _VB - 1) // _VB


def _tc_score(vn, sg, W3a, W3b, b3r, table):
    def body(vn_ref, sg_ref, w3a_ref, w3b_ref, b3_ref, tbl_ref, sh_ref, z_ref):
        @pl.when(pl.program_id(0) == 0)
        def _():
            sh_ref[...] = (
                lax.dot_general(vn_ref[...], w3a_ref[...], (((1,), (1,)), ((), ())),
                                preferred_element_type=jnp.float32)
                + lax.dot_general(sg_ref[...], w3b_ref[...], (((1,), (1,)), ((), ())),
                                  preferred_element_type=jnp.float32)
                + b3_ref[...]
            )

        z_ref[...] = lax.dot_general(sh_ref[...].astype(jnp.bfloat16),
                                     tbl_ref[...].astype(jnp.bfloat16),
                                     (((1,), (1,)), ((), ())),
                                     preferred_element_type=jnp.float32)

    return pl.pallas_call(
        body,
        grid=(_NVB,),
        in_specs=[
            pl.BlockSpec((NSEG, HID), lambda i: (0, 0)),
            pl.BlockSpec((NSEG, HID), lambda i: (0, 0)),
            pl.BlockSpec((HID, HID), lambda i: (0, 0)),
            pl.BlockSpec((HID, HID), lambda i: (0, 0)),
            pl.BlockSpec((1, HID), lambda i: (0, 0)),
            pl.BlockSpec((_VB, HID), lambda i: (i, 0)),
        ],
        out_specs=[
            pl.BlockSpec((NSEG, HID), lambda i: (0, 0)),
            pl.BlockSpec((NSEG, _VB), lambda i: (0, i)),
        ],
        out_shape=[
            jax.ShapeDtypeStruct((NSEG, HID), jnp.float32),
            jax.ShapeDtypeStruct((NSEG, NVOC), jnp.float32),
        ],
    )(vn, sg, W3a, W3b, b3r, table)


def kernel(session_embedding, batch, all_item_embedding, W1, b1, W2, b2, Wq, bq, W3, b3):
    batch = batch.astype(jnp.int32)
    batch_ext = jnp.concatenate([batch, jnp.full((_KR,), NSEG, jnp.int32)])

    batch2d = batch.reshape(N_TOK // _TB, 1, _TB)
    partials = _sc_boundaries(batch_ext)
    vn, lp = _sc_gather_vn(partials, session_embedding)
    wqp = jnp.pad(Wq.reshape(HID, 1).astype(jnp.bfloat16), ((0, 0), (0, 127)))
    ae_pad = _tc_alpha_ae(session_embedding, batch2d, vn, W1,
                          (b1 + b2).reshape(1, HID), W2, wqp, bq.reshape(1, 1))
    sg = _sc_segment_sum(batch_ext, lp, ae_pad)
    sh, z = _tc_score(vn, sg, W3[:, :HID], W3[:, HID:],
                      b3.reshape(1, HID), all_item_embedding)
    return sh, z


# speedup vs baseline: 1.6367x; 1.0040x over previous
import functools

import jax
import jax.numpy as jnp
from jax import lax
from jax.experimental import pallas as pl
from jax.experimental.pallas import tpu as pltpu
from jax.experimental.pallas import tpu_sc as plsc

N_TOK = 102400
NSEG = 1024
HID = 256
NVOC = 100000

_NC, _NS, _L = 2, 16, 16
_NW = _NC * _NS
_CHUNK = N_TOK // _NW
_NVEC = _CHUNK // _L
_SEGW = NSEG // _NW
_KR = 128
_JC = _CHUNK // _KR
_ZR = NSEG // _NS

@functools.cache
def _mesh():
    return plsc.VectorSubcoreMesh(core_axis_name="c", subcore_axis_name="s",
                                  num_cores=_NC, num_subcores=_NS)


def _worker_id():
    return lax.axis_index("s") * _NC + lax.axis_index("c")


def _sc_boundaries(batch_ext):
    @functools.partial(
        pl.kernel,
        out_type=jax.ShapeDtypeStruct((_NW * NSEG,), jnp.int32),
        mesh=_mesh(),
        compiler_params=pltpu.CompilerParams(needs_layout_passes=False),
        scratch_types=[
            pltpu.VMEM((_CHUNK + _L,), jnp.int32),
            pltpu.VMEM((NSEG,), jnp.int32),
        ],
    )
    def k(batch_hbm, out_hbm, bvec, last_local):
        wid = _worker_id()
        base = wid * _CHUNK
        pltpu.sync_copy(batch_hbm.at[pl.ds(base, _CHUNK + _L)], bvec)
        zero = jnp.zeros((_L,), jnp.int32)

        def zbody(i, c):
            last_local[pl.ds(i * _L, _L)] = zero
            return c

        lax.fori_loop(0, NSEG // _L, zbody, 0)
        basev = lax.iota(jnp.int32, _L) + (base + 1)

        def body(j, c):
            cur = bvec[pl.ds(j * _L, _L)]
            nxt = bvec[pl.ds(j * _L + 1, _L)]
            vals = basev + j * _L
            plsc.store_scatter(last_local, [cur], vals, mask=cur != nxt)
            return c

        lax.fori_loop(0, _NVEC, body, 0)
        pltpu.sync_copy(last_local, out_hbm.at[pl.ds(wid * NSEG, NSEG)])

    return k(batch_ext)


def _sc_gather_vn(partials, emb):
    @functools.partial(
        pl.kernel,
        out_type=(jax.ShapeDtypeStruct((NSEG, HID), jnp.float32),
                  jax.ShapeDtypeStruct((NSEG,), jnp.int32)),
        mesh=_mesh(),
        compiler_params=pltpu.CompilerParams(needs_layout_passes=False),
        scratch_types=[
            pltpu.VMEM((_NW, _SEGW), jnp.int32),
            pltpu.VMEM((_SEGW,), jnp.int32),
            pltpu.VMEM((_SEGW,), jnp.int32),
            pltpu.VMEM((_SEGW, HID), jnp.float32),
            pltpu.SemaphoreType.DMA,
        ],
    )
    def k(part_hbm, emb_hbm, vn_hbm, lp_hbm, pm, idxv, lpv, rows, sem):
        wid = _worker_id()
        for j in range(_NW):
            pltpu.sync_copy(part_hbm.at[pl.ds(j * NSEG + wid * _SEGW, _SEGW)],
                            pm.at[j])
        acc0 = pm[0, pl.ds(0, _L)]
        acc1 = pm[0, pl.ds(_L, _L)]
        for j in range(1, _NW):
            acc0 = jnp.maximum(acc0, pm[j, pl.ds(0, _L)])
            acc1 = jnp.maximum(acc1, pm[j, pl.ds(_L, _L)])
        lpv[pl.ds(0, _L)] = acc0
        lpv[pl.ds(_L, _L)] = acc1
        idxv[pl.ds(0, _L)] = jnp.maximum(acc0 - 1, 0)
        idxv[pl.ds(_L, _L)] = jnp.maximum(acc1 - 1, 0)
        pltpu.async_copy(emb_hbm.at[idxv], rows, sem).wait()
        pltpu.sync_copy(rows, vn_hbm.at[pl.ds(wid * _SEGW, _SEGW)])
        pltpu.sync_copy(lpv, lp_hbm.at[pl.ds(wid * _SEGW, _SEGW)])

    return k(partials, emb)


_TRASH = _SEGW


def _sc_segment_sum(batch_ext, lp, ae_pad):
    @functools.partial(
        pl.kernel,
        out_type=jax.ShapeDtypeStruct((NSEG, HID), jnp.float32),
        mesh=_mesh(),
        compiler_params=pltpu.CompilerParams(needs_layout_passes=False),
        scratch_types=[
            pltpu.VMEM((NSEG,), jnp.int32),
            pltpu.VMEM((_KR,), jnp.int32),
            pltpu.VMEM((_KR, HID // 2), jnp.uint32),
            pltpu.VMEM((_KR, HID // 2), jnp.uint32),
            pltpu.VMEM((_SEGW + 8, HID), jnp.float32),
            pltpu.SemaphoreType.DMA,
            pltpu.SemaphoreType.DMA,
        ],
    )
    def k(bext_hbm, lp_hbm, ae_hbm, out_hbm, lpv, bv, rows0, rows1, acc, sem0, sem1):
        wid = _worker_id()
        sbase = wid * _SEGW
        pltpu.sync_copy(lp_hbm, lpv)
        zero = jnp.zeros((_L,), jnp.float32)

        def zrow(i, c):
            for q in range(HID // _L):
                acc[i, pl.ds(q * _L, _L)] = zero
            return c

        lax.fori_loop(0, _SEGW + 8, zrow, 0)

        def pbody(v, carry):
            m, s, e = carry
            s = jnp.where(v * _L == sbase, m, s)
            e = jnp.where(v * _L == sbase + _SEGW, m, e)
            m = jnp.maximum(m, jnp.max(lpv[pl.ds(v * _L, _L)]))
            return m, s, e

        m, start, end = lax.fori_loop(0, NSEG // _L, pbody,
                                      (jnp.int32(0), jnp.int32(0), jnp.int32(0)))
        end = jnp.where(sbase + _SEGW == NSEG, m, end)
        astart = (start // 8) * 8
        nch = (end - astart + _KR - 1) // _KR

        hmask = jnp.full((_L,), 0xFFFF0000, jnp.uint32)

        def cvt(u):
            fe = plsc.bitcast(u << 16, jnp.float32)
            fo = plsc.bitcast(u & hmask, jnp.float32)
            return fe, fo

        def accum(rows):
            def vbody(v, c):
                lvec = bv[pl.ds(v * _L, _L)] - sbase
                l0 = lvec[0]
                l15 = lvec[_L - 1]
                r0 = v * _L
                uni = (l0 == l15) & (l0 >= 0) & (l0 < _SEGW)

                @pl.when(uni)
                def _():
                    for j in range(HID // 32):
                        sl = pl.ds(j * _L, _L)
                        es = []
                        os_ = []
                        for t in range(_L):
                            fe, fo = cvt(rows[r0 + t, sl])
                            es.append(fe)
                            os_.append(fo)
                        while len(es) > 1:
                            es = [es[i] + es[i + 1] for i in range(0, len(es), 2)]
                            os_ = [os_[i] + os_[i + 1] for i in range(0, len(os_), 2)]
                        plsc.addupdate(acc.at[l0, pl.ds(j * _L, _L)], es[0])
                        plsc.addupdate(acc.at[l0, pl.ds(128 + j * _L, _L)], os_[0])

                @pl.when(jnp.logical_not(uni))
                def _():
                    for lane in range(_L):
                        lv = lvec[lane]
                        lv = jnp.where((lv >= 0) & (lv < _SEGW), lv, _TRASH)
                        for j in range(HID // 32):
                            fe, fo = cvt(rows[r0 + lane, pl.ds(j * _L, _L)])
                            plsc.addupdate(acc.at[lv, pl.ds(j * _L, _L)], fe)
                            plsc.addupdate(acc.at[lv, pl.ds(128 + j * _L, _L)], fo)
                return c

            lax.fori_loop(0, _KR // _L, vbody, 0)

        @pl.when(nch > 0)
        def _():
            pltpu.async_copy(ae_hbm.at[pl.ds(astart, _KR)], rows0, sem0)

        def body(i, carry):
            c0 = 2 * i
            c1 = c0 + 1
            c2 = c0 + 2
            off0 = astart + c0 * _KR

            @pl.when(c1 < nch)
            def _():
                pltpu.async_copy(ae_hbm.at[pl.ds(off0 + _KR, _KR)], rows1, sem1)

            pltpu.sync_copy(bext_hbm.at[pl.ds(off0, _KR)], bv)
            pltpu.make_async_copy(ae_hbm.at[pl.ds(off0, _KR)], rows0, sem0).wait()
            accum(rows0)

            @pl.when(c2 < nch)
            def _():
                pltpu.async_copy(ae_hbm.at[pl.ds(off0 + 2 * _KR, _KR)], rows0, sem0)

            @pl.when(c1 < nch)
            def _():
                pltpu.sync_copy(bext_hbm.at[pl.ds(off0 + _KR, _KR)], bv)
                pltpu.make_async_copy(ae_hbm.at[pl.ds(off0 + _KR, _KR)], rows1,
                                      sem1).wait()
                accum(rows1)

            return carry

        lax.fori_loop(0, (nch + 1) // 2, body, 0)
        pltpu.sync_copy(acc.at[pl.ds(0, _SEGW)], out_hbm.at[pl.ds(sbase, _SEGW)])

    return k(batch_ext, lp, ae_pad)


_TB = 1024


def _tc_alpha_ae(emb, batch2d, vn, W1, bias, W2, Wqp, bq2):
    def body(emb_ref, b_ref, vn_ref, w1_ref, bias_ref, w2_ref, wqp_ref, bq_ref,
             out_ref, t1_ref):
        @pl.when(pl.program_id(0) == 0)
        def _():
            t1_ref[...] = (
                lax.dot_general(vn_ref[...], w1_ref[...], (((1,), (1,)), ((), ())),
                                preferred_element_type=jnp.float32)
                + bias_ref[...]
            )

        t2 = lax.dot_general(emb_ref[...].astype(jnp.bfloat16),
                             w2_ref[...].astype(jnp.bfloat16),
                             (((1,), (1,)), ((), ())),
                             preferred_element_type=jnp.float32)
        ids = b_ref[0, 0].astype(jnp.int16)
        cols = lax.broadcasted_iota(jnp.int16, (_TB, NSEG), 1)
        oh = (ids.reshape(_TB, 1) == cols).astype(jnp.bfloat16)
        t1x = lax.dot_general(oh, t1_ref[...].astype(jnp.bfloat16),
                              (((1,), (0,)), ((), ())),
                              preferred_element_type=jnp.float32)
        pre = t2 + t1x
        sg = 1.0 / (1.0 + jnp.exp(-pre))
        af = lax.dot_general(sg.astype(jnp.bfloat16), wqp_ref[...],
                             (((1,), (0,)), ((), ())),
                             preferred_element_type=jnp.float32)
        alpha = af[:, 0:1] + bq_ref[...]
        ab = alpha * emb_ref[...]
        lo = lax.bitcast_convert_type(ab[:, :128].astype(jnp.bfloat16),
                                      jnp.uint16).astype(jnp.uint32)
        hi = lax.bitcast_convert_type(ab[:, 128:].astype(jnp.bfloat16),
                                      jnp.uint16).astype(jnp.uint32)
        out_ref[...] = (hi << 16) | lo

    return pl.pallas_call(
        body,
        grid=(N_TOK // _TB,),
        in_specs=[
            pl.BlockSpec((_TB, HID), lambda i: (i, 0)),
            pl.BlockSpec((1, 1, _TB), lambda i: (i, 0, 0)),
            pl.BlockSpec((NSEG, HID), lambda i: (0, 0)),
            pl.BlockSpec((HID, HID), lambda i: (0, 0)),
            pl.BlockSpec((1, HID), lambda i: (0, 0)),
            pl.BlockSpec((HID, HID), lambda i: (0, 0)),
            pl.BlockSpec((HID, 128), lambda i: (0, 0)),
            pl.BlockSpec((1, 1), lambda i: (0, 0)),
        ],
        out_specs=pl.BlockSpec((_TB, HID // 2), lambda i: (i, 0)),
        out_shape=jax.ShapeDtypeStruct((N_TOK + _KR, HID // 2), jnp.uint32),
        scratch_shapes=[pltpu.VMEM((NSEG, HID), jnp.float32)],
    )(emb, batch2d, vn, W1, bias, W2, Wqp, bq2)


_VB = 4096
_NVB = (NVOC + _VB - 1) // _VB


def _tc_score(vn, sg, W3a, W3b, b3r, table):
    def body(vn_ref, sg_ref, w3a_ref, w3b_ref, b3_ref, tbl_ref, sh_ref, z_ref):
        @pl.when(pl.program_id(0) == 0)
        def _():
            sh_ref[...] = (
                lax.dot_general(vn_ref[...], w3a_ref[...], (((1,), (1,)), ((), ())),
                                preferred_element_type=jnp.float32)
                + lax.dot_general(sg_ref[...], w3b_ref[...], (((1,), (1,)), ((), ())),
                                  preferred_element_type=jnp.float32)
                + b3_ref[...]
            )

        z_ref[...] = lax.dot_general(sh_ref[...].astype(jnp.bfloat16),
                                     tbl_ref[...].astype(jnp.bfloat16),
                                     (((1,), (1,)), ((), ())),
                                     preferred_element_type=jnp.float32)

    return pl.pallas_call(
        body,
        grid=(_NVB,),
        in_specs=[
            pl.BlockSpec((NSEG, HID), lambda i: (0, 0)),
            pl.BlockSpec((NSEG, HID), lambda i: (0, 0)),
            pl.BlockSpec((HID, HID), lambda i: (0, 0)),
            pl.BlockSpec((HID, HID), lambda i: (0, 0)),
            pl.BlockSpec((1, HID), lambda i: (0, 0)),
            pl.BlockSpec((_VB, HID), lambda i: (i, 0)),
        ],
        out_specs=[
            pl.BlockSpec((NSEG, HID), lambda i: (0, 0)),
            pl.BlockSpec((NSEG, _VB), lambda i: (0, i)),
        ],
        out_shape=[
            jax.ShapeDtypeStruct((NSEG, HID), jnp.float32),
            jax.ShapeDtypeStruct((NSEG, NVOC), jnp.float32),
        ],
    )(vn, sg, W3a, W3b, b3r, table)


def kernel(session_embedding, batch, all_item_embedding, W1, b1, W2, b2, Wq, bq, W3, b3):
    batch = batch.astype(jnp.int32)
    batch_ext = jnp.concatenate([batch, jnp.full((_KR,), NSEG, jnp.int32)])

    batch2d = batch.reshape(N_TOK // _TB, 1, _TB)
    partials = _sc_boundaries(batch_ext)
    vn, lp = _sc_gather_vn(partials, session_embedding)
    wqp = jnp.pad(Wq.reshape(HID, 1).astype(jnp.bfloat16), ((0, 0), (0, 127)))
    ae_pad = _tc_alpha_ae(session_embedding, batch2d, vn, W1,
                          (b1 + b2).reshape(1, HID), W2, wqp, bq.reshape(1, 1))
    sg = _sc_segment_sum(batch_ext, lp, ae_pad)
    sh, z = _tc_score(vn, sg, W3[:, :HID], W3[:, HID:],
                      b3.reshape(1, HID), all_item_embedding)
    return sh, z


# speedup vs baseline: 1.6781x; 1.0253x over previous
import functools

import jax
import jax.numpy as jnp
from jax import lax
from jax.experimental import pallas as pl
from jax.experimental.pallas import tpu as pltpu
from jax.experimental.pallas import tpu_sc as plsc

N_TOK = 102400
NSEG = 1024
HID = 256
NVOC = 100000

_NC, _NS, _L = 2, 16, 16
_NW = _NC * _NS
_CHUNK = N_TOK // _NW
_NVEC = _CHUNK // _L
_SEGW = NSEG // _NW
_KR = 128
_JC = _CHUNK // _KR
_ZR = NSEG // _NS

@functools.cache
def _mesh():
    return plsc.VectorSubcoreMesh(core_axis_name="c", subcore_axis_name="s",
                                  num_cores=_NC, num_subcores=_NS)


def _worker_id():
    return lax.axis_index("s") * _NC + lax.axis_index("c")


def _sc_boundaries(batch_ext):
    @functools.partial(
        pl.kernel,
        out_type=jax.ShapeDtypeStruct((_NW * NSEG,), jnp.int32),
        mesh=_mesh(),
        compiler_params=pltpu.CompilerParams(needs_layout_passes=False),
        scratch_types=[
            pltpu.VMEM((_CHUNK + _L,), jnp.int32),
            pltpu.VMEM((NSEG,), jnp.int32),
        ],
    )
    def k(batch_hbm, out_hbm, bvec, last_local):
        wid = _worker_id()
        base = wid * _CHUNK
        pltpu.sync_copy(batch_hbm.at[pl.ds(base, _CHUNK + _L)], bvec)
        zero = jnp.zeros((_L,), jnp.int32)

        def zbody(i, c):
            last_local[pl.ds(i * _L, _L)] = zero
            return c

        lax.fori_loop(0, NSEG // _L, zbody, 0)
        basev = lax.iota(jnp.int32, _L) + (base + 1)

        def body(j, c):
            cur = bvec[pl.ds(j * _L, _L)]
            nxt = bvec[pl.ds(j * _L + 1, _L)]
            vals = basev + j * _L
            plsc.store_scatter(last_local, [cur], vals, mask=cur != nxt)
            return c

        lax.fori_loop(0, _NVEC, body, 0)
        pltpu.sync_copy(last_local, out_hbm.at[pl.ds(wid * NSEG, NSEG)])

    return k(batch_ext)


def _sc_gather_vn(partials, emb):
    @functools.partial(
        pl.kernel,
        out_type=(jax.ShapeDtypeStruct((NSEG, HID), jnp.float32),
                  jax.ShapeDtypeStruct((NSEG,), jnp.int32)),
        mesh=_mesh(),
        compiler_params=pltpu.CompilerParams(needs_layout_passes=False),
        scratch_types=[
            pltpu.VMEM((_NW, _SEGW), jnp.int32),
            pltpu.VMEM((_SEGW,), jnp.int32),
            pltpu.VMEM((_SEGW,), jnp.int32),
            pltpu.VMEM((_SEGW, HID), jnp.float32),
            pltpu.SemaphoreType.DMA,
        ],
    )
    def k(part_hbm, emb_hbm, vn_hbm, lp_hbm, pm, idxv, lpv, rows, sem):
        wid = _worker_id()
        for j in range(_NW):
            pltpu.sync_copy(part_hbm.at[pl.ds(j * NSEG + wid * _SEGW, _SEGW)],
                            pm.at[j])
        acc0 = pm[0, pl.ds(0, _L)]
        acc1 = pm[0, pl.ds(_L, _L)]
        for j in range(1, _NW):
            acc0 = jnp.maximum(acc0, pm[j, pl.ds(0, _L)])
            acc1 = jnp.maximum(acc1, pm[j, pl.ds(_L, _L)])
        lpv[pl.ds(0, _L)] = acc0
        lpv[pl.ds(_L, _L)] = acc1
        idxv[pl.ds(0, _L)] = jnp.maximum(acc0 - 1, 0)
        idxv[pl.ds(_L, _L)] = jnp.maximum(acc1 - 1, 0)
        pltpu.async_copy(emb_hbm.at[idxv], rows, sem).wait()
        pltpu.sync_copy(rows, vn_hbm.at[pl.ds(wid * _SEGW, _SEGW)])
        pltpu.sync_copy(lpv, lp_hbm.at[pl.ds(wid * _SEGW, _SEGW)])

    return k(partials, emb)


_TRASH = _SEGW


def _sc_segment_sum(batch_ext, lp, ae_pad):
    @functools.partial(
        pl.kernel,
        out_type=jax.ShapeDtypeStruct((NSEG, HID), jnp.float32),
        mesh=_mesh(),
        compiler_params=pltpu.CompilerParams(needs_layout_passes=False),
        scratch_types=[
            pltpu.VMEM((NSEG,), jnp.int32),
            pltpu.VMEM((_KR,), jnp.int32),
            pltpu.VMEM((_KR, HID // 2), jnp.uint32),
            pltpu.VMEM((_KR, HID // 2), jnp.uint32),
            pltpu.VMEM((_SEGW + 8, HID), jnp.float32),
            pltpu.SemaphoreType.DMA,
            pltpu.SemaphoreType.DMA,
        ],
    )
    def k(bext_hbm, lp_hbm, ae_hbm, out_hbm, lpv, bv, rows0, rows1, acc, sem0, sem1):
        wid = _worker_id()
        sbase = wid * _SEGW
        pltpu.sync_copy(lp_hbm, lpv)
        zero = jnp.zeros((_L,), jnp.float32)

        def zrow(i, c):
            for q in range(HID // _L):
                acc[i, pl.ds(q * _L, _L)] = zero
            return c

        lax.fori_loop(0, _SEGW + 8, zrow, 0)

        def pbody(v, carry):
            m, s, e = carry
            s = jnp.where(v * _L == sbase, m, s)
            e = jnp.where(v * _L == sbase + _SEGW, m, e)
            m = jnp.maximum(m, jnp.max(lpv[pl.ds(v * _L, _L)]))
            return m, s, e

        m, start, end = lax.fori_loop(0, NSEG // _L, pbody,
                                      (jnp.int32(0), jnp.int32(0), jnp.int32(0)))
        end = jnp.where(sbase + _SEGW == NSEG, m, end)
        astart = (start // 8) * 8
        nch = (end - astart + _KR - 1) // _KR

        hmask = jnp.full((_L,), 0xFFFF0000, jnp.uint32)

        def cvt(u):
            fe = plsc.bitcast(u << 16, jnp.float32)
            fo = plsc.bitcast(u & hmask, jnp.float32)
            return fe, fo

        def accum(rows):
            def vbody(v, c):
                lvec = bv[pl.ds(v * _L, _L)] - sbase
                l0 = lvec[0]
                l15 = lvec[_L - 1]
                r0 = v * _L
                uni = (l0 == l15) & (l0 >= 0) & (l0 < _SEGW)

                @pl.when(uni)
                def _():
                    for j in range(HID // 32):
                        sl = pl.ds(j * _L, _L)
                        es = []
                        os_ = []
                        for t in range(_L):
                            fe, fo = cvt(rows[r0 + t, sl])
                            es.append(fe)
                            os_.append(fo)
                        while len(es) > 1:
                            es = [es[i] + es[i + 1] for i in range(0, len(es), 2)]
                            os_ = [os_[i] + os_[i + 1] for i in range(0, len(os_), 2)]
                        plsc.addupdate(acc.at[l0, pl.ds(j * _L, _L)], es[0])
                        plsc.addupdate(acc.at[l0, pl.ds(128 + j * _L, _L)], os_[0])

                @pl.when(jnp.logical_not(uni))
                def _():
                    for lane in range(_L):
                        lv = lvec[lane]
                        lv = jnp.where((lv >= 0) & (lv < _SEGW), lv, _TRASH)
                        for j in range(HID // 32):
                            fe, fo = cvt(rows[r0 + lane, pl.ds(j * _L, _L)])
                            plsc.addupdate(acc.at[lv, pl.ds(j * _L, _L)], fe)
                            plsc.addupdate(acc.at[lv, pl.ds(128 + j * _L, _L)], fo)
                return c

            lax.fori_loop(0, _KR // _L, vbody, 0)

        @pl.when(nch > 0)
        def _():
            pltpu.async_copy(ae_hbm.at[pl.ds(astart, _KR)], rows0, sem0)

        def body(i, carry):
            c0 = 2 * i
            c1 = c0 + 1
            c2 = c0 + 2
            off0 = astart + c0 * _KR

            @pl.when(c1 < nch)
            def _():
                pltpu.async_copy(ae_hbm.at[pl.ds(off0 + _KR, _KR)], rows1, sem1)

            pltpu.sync_copy(bext_hbm.at[pl.ds(off0, _KR)], bv)
            pltpu.make_async_copy(ae_hbm.at[pl.ds(off0, _KR)], rows0, sem0).wait()
            accum(rows0)

            @pl.when(c2 < nch)
            def _():
                pltpu.async_copy(ae_hbm.at[pl.ds(off0 + 2 * _KR, _KR)], rows0, sem0)

            @pl.when(c1 < nch)
            def _():
                pltpu.sync_copy(bext_hbm.at[pl.ds(off0 + _KR, _KR)], bv)
                pltpu.make_async_copy(ae_hbm.at[pl.ds(off0 + _KR, _KR)], rows1,
                                      sem1).wait()
                accum(rows1)

            return carry

        lax.fori_loop(0, (nch + 1) // 2, body, 0)
        pltpu.sync_copy(acc.at[pl.ds(0, _SEGW)], out_hbm.at[pl.ds(sbase, _SEGW)])

    return k(batch_ext, lp, ae_pad)


_TB = 2048


def _tc_alpha_ae(emb, batch2d, vn, W1, bias, W2, Wqp, bq2):
    def body(emb_ref, b_ref, vn_ref, w1_ref, bias_ref, w2_ref, wqp_ref, bq_ref,
             out_ref, t1_ref):
        @pl.when(pl.program_id(0) == 0)
        def _():
            t1_ref[...] = (
                lax.dot_general(vn_ref[...], w1_ref[...], (((1,), (1,)), ((), ())),
                                preferred_element_type=jnp.float32)
                + bias_ref[...]
            )

        t2 = lax.dot_general(emb_ref[...].astype(jnp.bfloat16),
                             w2_ref[...].astype(jnp.bfloat16),
                             (((1,), (1,)), ((), ())),
                             preferred_element_type=jnp.float32)
        ids = b_ref[0, 0].astype(jnp.int16)
        cols = lax.broadcasted_iota(jnp.int16, (_TB, NSEG), 1)
        oh = (ids.reshape(_TB, 1) == cols).astype(jnp.bfloat16)
        t1x = lax.dot_general(oh, t1_ref[...].astype(jnp.bfloat16),
                              (((1,), (0,)), ((), ())),
                              preferred_element_type=jnp.float32)
        pre = t2 + t1x
        sg = 1.0 / (1.0 + jnp.exp(-pre))
        af = lax.dot_general(sg.astype(jnp.bfloat16), wqp_ref[...],
                             (((1,), (0,)), ((), ())),
                             preferred_element_type=jnp.float32)
        alpha = af[:, 0:1] + bq_ref[...]
        ab = alpha * emb_ref[...]
        lo = lax.bitcast_convert_type(ab[:, :128].astype(jnp.bfloat16),
                                      jnp.uint16).astype(jnp.uint32)
        hi = lax.bitcast_convert_type(ab[:, 128:].astype(jnp.bfloat16),
                                      jnp.uint16).astype(jnp.uint32)
        out_ref[...] = (hi << 16) | lo

    return pl.pallas_call(
        body,
        grid=(N_TOK // _TB,),
        in_specs=[
            pl.BlockSpec((_TB, HID), lambda i: (i, 0)),
            pl.BlockSpec((1, 1, _TB), lambda i: (i, 0, 0)),
            pl.BlockSpec((NSEG, HID), lambda i: (0, 0)),
            pl.BlockSpec((HID, HID), lambda i: (0, 0)),
            pl.BlockSpec((1, HID), lambda i: (0, 0)),
            pl.BlockSpec((HID, HID), lambda i: (0, 0)),
            pl.BlockSpec((HID, 128), lambda i: (0, 0)),
            pl.BlockSpec((1, 1), lambda i: (0, 0)),
        ],
        out_specs=pl.BlockSpec((_TB, HID // 2), lambda i: (i, 0)),
        out_shape=jax.ShapeDtypeStruct((N_TOK + _KR, HID // 2), jnp.uint32),
        scratch_shapes=[pltpu.VMEM((NSEG, HID), jnp.float32)],
    )(emb, batch2d, vn, W1, bias, W2, Wqp, bq2)


_VB = 4096
_NVB = (NVOC + _VB - 1) // _VB


def _tc_score(vn, sg, W3a, W3b, b3r, table):
    def body(vn_ref, sg_ref, w3a_ref, w3b_ref, b3_ref, tbl_ref, sh_ref, z_ref):
        @pl.when(pl.program_id(0) == 0)
        def _():
            sh_ref[...] = (
                lax.dot_general(vn_ref[...], w3a_ref[...], (((1,), (1,)), ((), ())),
                                preferred_element_type=jnp.float32)
                + lax.dot_general(sg_ref[...], w3b_ref[...], (((1,), (1,)), ((), ())),
                                  preferred_element_type=jnp.float32)
                + b3_ref[...]
            )

        z_ref[...] = lax.dot_general(sh_ref[...].astype(jnp.bfloat16),
                                     tbl_ref[...].astype(jnp.bfloat16),
                                     (((1,), (1,)), ((), ())),
                                     preferred_element_type=jnp.float32)

    return pl.pallas_call(
        body,
        grid=(_NVB,),
        in_specs=[
            pl.BlockSpec((NSEG, HID), lambda i: (0, 0)),
            pl.BlockSpec((NSEG, HID), lambda i: (0, 0)),
            pl.BlockSpec((HID, HID), lambda i: (0, 0)),
            pl.BlockSpec((HID, HID), lambda i: (0, 0)),
            pl.BlockSpec((1, HID), lambda i: (0, 0)),
            pl.BlockSpec((_VB, HID), lambda i: (i, 0)),
        ],
        out_specs=[
            pl.BlockSpec((NSEG, HID), lambda i: (0, 0)),
            pl.BlockSpec((NSEG, _VB), lambda i: (0, i)),
        ],
        out_shape=[
            jax.ShapeDtypeStruct((NSEG, HID), jnp.float32),
            jax.ShapeDtypeStruct((NSEG, NVOC), jnp.float32),
        ],
    )(vn, sg, W3a, W3b, b3r, table)


def kernel(session_embedding, batch, all_item_embedding, W1, b1, W2, b2, Wq, bq, W3, b3):
    batch = batch.astype(jnp.int32)
    batch_ext = jnp.concatenate([batch, jnp.full((_KR,), NSEG, jnp.int32)])

    batch2d = batch.reshape(N_TOK // _TB, 1, _TB)
    partials = _sc_boundaries(batch_ext)
    vn, lp = _sc_gather_vn(partials, session_embedding)
    wqp = jnp.pad(Wq.reshape(HID, 1).astype(jnp.bfloat16), ((0, 0), (0, 127)))
    ae_pad = _tc_alpha_ae(session_embedding, batch2d, vn, W1,
                          (b1 + b2).reshape(1, HID), W2, wqp, bq.reshape(1, 1))
    sg = _sc_segment_sum(batch_ext, lp, ae_pad)
    sh, z = _tc_score(vn, sg, W3[:, :HID], W3[:, HID:],
                      b3.reshape(1, HID), all_item_embedding)
    return sh, z


# speedup vs baseline: 1.6926x; 1.0086x over previous
import functools

import jax
import jax.numpy as jnp
from jax import lax
from jax.experimental import pallas as pl
from jax.experimental.pallas import tpu as pltpu
from jax.experimental.pallas import tpu_sc as plsc

N_TOK = 102400
NSEG = 1024
HID = 256
NVOC = 100000

_NC, _NS, _L = 2, 16, 16
_NW = _NC * _NS
_CHUNK = N_TOK // _NW
_NVEC = _CHUNK // _L
_SEGW = NSEG // _NW
_KR = 128
_JC = _CHUNK // _KR
_ZR = NSEG // _NS

@functools.cache
def _mesh():
    return plsc.VectorSubcoreMesh(core_axis_name="c", subcore_axis_name="s",
                                  num_cores=_NC, num_subcores=_NS)


def _worker_id():
    return lax.axis_index("s") * _NC + lax.axis_index("c")


def _sc_boundaries(batch_ext):
    @functools.partial(
        pl.kernel,
        out_type=jax.ShapeDtypeStruct((_NW * NSEG,), jnp.int32),
        mesh=_mesh(),
        compiler_params=pltpu.CompilerParams(needs_layout_passes=False),
        scratch_types=[
            pltpu.VMEM((_CHUNK + _L,), jnp.int32),
            pltpu.VMEM((NSEG,), jnp.int32),
        ],
    )
    def k(batch_hbm, out_hbm, bvec, last_local):
        wid = _worker_id()
        base = wid * _CHUNK
        pltpu.sync_copy(batch_hbm.at[pl.ds(base, _CHUNK + _L)], bvec)
        zero = jnp.zeros((_L,), jnp.int32)

        def zbody(i, c):
            last_local[pl.ds(i * _L, _L)] = zero
            return c

        lax.fori_loop(0, NSEG // _L, zbody, 0)
        basev = lax.iota(jnp.int32, _L) + (base + 1)

        def body(j, c):
            cur = bvec[pl.ds(j * _L, _L)]
            nxt = bvec[pl.ds(j * _L + 1, _L)]
            vals = basev + j * _L
            plsc.store_scatter(last_local, [cur], vals, mask=cur != nxt)
            return c

        lax.fori_loop(0, _NVEC, body, 0)
        pltpu.sync_copy(last_local, out_hbm.at[pl.ds(wid * NSEG, NSEG)])

    return k(batch_ext)


def _sc_gather_vn(partials, emb):
    @functools.partial(
        pl.kernel,
        out_type=(jax.ShapeDtypeStruct((NSEG, HID), jnp.float32),
                  jax.ShapeDtypeStruct((NSEG,), jnp.int32)),
        mesh=_mesh(),
        compiler_params=pltpu.CompilerParams(needs_layout_passes=False),
        scratch_types=[
            pltpu.VMEM((_NW, _SEGW), jnp.int32),
            pltpu.VMEM((_SEGW,), jnp.int32),
            pltpu.VMEM((_SEGW,), jnp.int32),
            pltpu.VMEM((_SEGW, HID), jnp.float32),
            pltpu.SemaphoreType.DMA,
        ],
    )
    def k(part_hbm, emb_hbm, vn_hbm, lp_hbm, pm, idxv, lpv, rows, sem):
        wid = _worker_id()
        for j in range(_NW):
            pltpu.sync_copy(part_hbm.at[pl.ds(j * NSEG + wid * _SEGW, _SEGW)],
                            pm.at[j])
        acc0 = pm[0, pl.ds(0, _L)]
        acc1 = pm[0, pl.ds(_L, _L)]
        for j in range(1, _NW):
            acc0 = jnp.maximum(acc0, pm[j, pl.ds(0, _L)])
            acc1 = jnp.maximum(acc1, pm[j, pl.ds(_L, _L)])
        lpv[pl.ds(0, _L)] = acc0
        lpv[pl.ds(_L, _L)] = acc1
        idxv[pl.ds(0, _L)] = jnp.maximum(acc0 - 1, 0)
        idxv[pl.ds(_L, _L)] = jnp.maximum(acc1 - 1, 0)
        pltpu.async_copy(emb_hbm.at[idxv], rows, sem).wait()
        pltpu.sync_copy(rows, vn_hbm.at[pl.ds(wid * _SEGW, _SEGW)])
        pltpu.sync_copy(lpv, lp_hbm.at[pl.ds(wid * _SEGW, _SEGW)])

    return k(partials, emb)


_TRASH = _SEGW


def _sc_segment_sum(batch_ext, lp, ae_pad):
    @functools.partial(
        pl.kernel,
        out_type=jax.ShapeDtypeStruct((NSEG, HID), jnp.float32),
        mesh=_mesh(),
        compiler_params=pltpu.CompilerParams(needs_layout_passes=False),
        scratch_types=[
            pltpu.VMEM((NSEG,), jnp.int32),
            pltpu.VMEM((_KR,), jnp.int32),
            pltpu.VMEM((_KR, HID // 2), jnp.uint32),
            pltpu.VMEM((_KR, HID // 2), jnp.uint32),
            pltpu.VMEM((_SEGW + 8, HID), jnp.float32),
            pltpu.SemaphoreType.DMA,
            pltpu.SemaphoreType.DMA,
        ],
    )
    def k(bext_hbm, lp_hbm, ae_hbm, out_hbm, lpv, bv, rows0, rows1, acc, sem0, sem1):
        wid = _worker_id()
        sbase = wid * _SEGW
        pltpu.sync_copy(lp_hbm, lpv)
        zero = jnp.zeros((_L,), jnp.float32)

        def zrow(i, c):
            for q in range(HID // _L):
                acc[i, pl.ds(q * _L, _L)] = zero
            return c

        lax.fori_loop(0, _SEGW + 8, zrow, 0)

        def pbody(v, carry):
            m, s, e = carry
            s = jnp.where(v * _L == sbase, m, s)
            e = jnp.where(v * _L == sbase + _SEGW, m, e)
            m = jnp.maximum(m, jnp.max(lpv[pl.ds(v * _L, _L)]))
            return m, s, e

        m, start, end = lax.fori_loop(0, NSEG // _L, pbody,
                                      (jnp.int32(0), jnp.int32(0), jnp.int32(0)))
        end = jnp.where(sbase + _SEGW == NSEG, m, end)
        astart = (start // 8) * 8
        nch = (end - astart + _KR - 1) // _KR

        hmask = jnp.full((_L,), 0xFFFF0000, jnp.uint32)

        def cvt(u):
            fe = plsc.bitcast(u << 16, jnp.float32)
            fo = plsc.bitcast(u & hmask, jnp.float32)
            return fe, fo

        def accum(rows):
            def vbody(v, c):
                lvec = bv[pl.ds(v * _L, _L)] - sbase
                l0 = lvec[0]
                l15 = lvec[_L - 1]
                r0 = v * _L
                uni = (l0 == l15) & (l0 >= 0) & (l0 < _SEGW)

                @pl.when(uni)
                def _():
                    for j in range(HID // 32):
                        sl = pl.ds(j * _L, _L)
                        es = []
                        os_ = []
                        for t in range(_L):
                            fe, fo = cvt(rows[r0 + t, sl])
                            es.append(fe)
                            os_.append(fo)
                        while len(es) > 1:
                            es = [es[i] + es[i + 1] for i in range(0, len(es), 2)]
                            os_ = [os_[i] + os_[i + 1] for i in range(0, len(os_), 2)]
                        plsc.addupdate(acc.at[l0, pl.ds(j * _L, _L)], es[0])
                        plsc.addupdate(acc.at[l0, pl.ds(128 + j * _L, _L)], os_[0])

                @pl.when(jnp.logical_not(uni))
                def _():
                    for lane in range(_L):
                        lv = lvec[lane]
                        lv = jnp.where((lv >= 0) & (lv < _SEGW), lv, _TRASH)
                        for j in range(HID // 32):
                            fe, fo = cvt(rows[r0 + lane, pl.ds(j * _L, _L)])
                            plsc.addupdate(acc.at[lv, pl.ds(j * _L, _L)], fe)
                            plsc.addupdate(acc.at[lv, pl.ds(128 + j * _L, _L)], fo)
                return c

            lax.fori_loop(0, _KR // _L, vbody, 0)

        @pl.when(nch > 0)
        def _():
            pltpu.async_copy(ae_hbm.at[pl.ds(astart, _KR)], rows0, sem0)

        def body(i, carry):
            c0 = 2 * i
            c1 = c0 + 1
            c2 = c0 + 2
            off0 = astart + c0 * _KR

            @pl.when(c1 < nch)
            def _():
                pltpu.async_copy(ae_hbm.at[pl.ds(off0 + _KR, _KR)], rows1, sem1)

            pltpu.sync_copy(bext_hbm.at[pl.ds(off0, _KR)], bv)
            pltpu.make_async_copy(ae_hbm.at[pl.ds(off0, _KR)], rows0, sem0).wait()
            accum(rows0)

            @pl.when(c2 < nch)
            def _():
                pltpu.async_copy(ae_hbm.at[pl.ds(off0 + 2 * _KR, _KR)], rows0, sem0)

            @pl.when(c1 < nch)
            def _():
                pltpu.sync_copy(bext_hbm.at[pl.ds(off0 + _KR, _KR)], bv)
                pltpu.make_async_copy(ae_hbm.at[pl.ds(off0 + _KR, _KR)], rows1,
                                      sem1).wait()
                accum(rows1)

            return carry

        lax.fori_loop(0, (nch + 1) // 2, body, 0)
        pltpu.sync_copy(acc.at[pl.ds(0, _SEGW)], out_hbm.at[pl.ds(sbase, _SEGW)])

    return k(batch_ext, lp, ae_pad)


_TB = 4096


def _tc_alpha_ae(emb, batch2d, vn, W1, bias, W2, Wqp, bq2):
    def body(emb_ref, b_ref, vn_ref, w1_ref, bias_ref, w2_ref, wqp_ref, bq_ref,
             out_ref, t1_ref):
        @pl.when(pl.program_id(0) == 0)
        def _():
            t1_ref[...] = (
                lax.dot_general(vn_ref[...], w1_ref[...], (((1,), (1,)), ((), ())),
                                preferred_element_type=jnp.float32)
                + bias_ref[...]
            )

        t2 = lax.dot_general(emb_ref[...].astype(jnp.bfloat16),
                             w2_ref[...].astype(jnp.bfloat16),
                             (((1,), (1,)), ((), ())),
                             preferred_element_type=jnp.float32)
        ids = b_ref[0, 0].astype(jnp.int16)
        cols = lax.broadcasted_iota(jnp.int16, (_TB, NSEG), 1)
        oh = (ids.reshape(_TB, 1) == cols).astype(jnp.bfloat16)
        t1x = lax.dot_general(oh, t1_ref[...].astype(jnp.bfloat16),
                              (((1,), (0,)), ((), ())),
                              preferred_element_type=jnp.float32)
        pre = t2 + t1x
        sg = 1.0 / (1.0 + jnp.exp(-pre))
        af = lax.dot_general(sg.astype(jnp.bfloat16), wqp_ref[...],
                             (((1,), (0,)), ((), ())),
                             preferred_element_type=jnp.float32)
        alpha = af[:, 0:1] + bq_ref[...]
        ab = alpha * emb_ref[...]
        lo = lax.bitcast_convert_type(ab[:, :128].astype(jnp.bfloat16),
                                      jnp.uint16).astype(jnp.uint32)
        hi = lax.bitcast_convert_type(ab[:, 128:].astype(jnp.bfloat16),
                                      jnp.uint16).astype(jnp.uint32)
        out_ref[...] = (hi << 16) | lo

    return pl.pallas_call(
        body,
        grid=(N_TOK // _TB,),
        in_specs=[
            pl.BlockSpec((_TB, HID), lambda i: (i, 0)),
            pl.BlockSpec((1, 1, _TB), lambda i: (i, 0, 0)),
            pl.BlockSpec((NSEG, HID), lambda i: (0, 0)),
            pl.BlockSpec((HID, HID), lambda i: (0, 0)),
            pl.BlockSpec((1, HID), lambda i: (0, 0)),
            pl.BlockSpec((HID, HID), lambda i: (0, 0)),
            pl.BlockSpec((HID, 128), lambda i: (0, 0)),
            pl.BlockSpec((1, 1), lambda i: (0, 0)),
        ],
        out_specs=pl.BlockSpec((_TB, HID // 2), lambda i: (i, 0)),
        out_shape=jax.ShapeDtypeStruct((N_TOK + _KR, HID // 2), jnp.uint32),
        scratch_shapes=[pltpu.VMEM((NSEG, HID), jnp.float32)],
    )(emb, batch2d, vn, W1, bias, W2, Wqp, bq2)


_VB = 4096
_NVB = (NVOC + _VB - 1) // _VB


def _tc_score(vn, sg, W3a, W3b, b3r, table):
    def body(vn_ref, sg_ref, w3a_ref, w3b_ref, b3_ref, tbl_ref, sh_ref, z_ref):
        @pl.when(pl.program_id(0) == 0)
        def _():
            sh_ref[...] = (
                lax.dot_general(vn_ref[...], w3a_ref[...], (((1,), (1,)), ((), ())),
                                preferred_element_type=jnp.float32)
                + lax.dot_general(sg_ref[...], w3b_ref[...], (((1,), (1,)), ((), ())),
                                  preferred_element_type=jnp.float32)
                + b3_ref[...]
            )

        z_ref[...] = lax.dot_general(sh_ref[...].astype(jnp.bfloat16),
                                     tbl_ref[...].astype(jnp.bfloat16),
                                     (((1,), (1,)), ((), ())),
                                     preferred_element_type=jnp.float32)

    return pl.pallas_call(
        body,
        grid=(_NVB,),
        in_specs=[
            pl.BlockSpec((NSEG, HID), lambda i: (0, 0)),
            pl.BlockSpec((NSEG, HID), lambda i: (0, 0)),
            pl.BlockSpec((HID, HID), lambda i: (0, 0)),
            pl.BlockSpec((HID, HID), lambda i: (0, 0)),
            pl.BlockSpec((1, HID), lambda i: (0, 0)),
            pl.BlockSpec((_VB, HID), lambda i: (i, 0)),
        ],
        out_specs=[
            pl.BlockSpec((NSEG, HID), lambda i: (0, 0)),
            pl.BlockSpec((NSEG, _VB), lambda i: (0, i)),
        ],
        out_shape=[
            jax.ShapeDtypeStruct((NSEG, HID), jnp.float32),
            jax.ShapeDtypeStruct((NSEG, NVOC), jnp.float32),
        ],
    )(vn, sg, W3a, W3b, b3r, table)


def kernel(session_embedding, batch, all_item_embedding, W1, b1, W2, b2, Wq, bq, W3, b3):
    batch = batch.astype(jnp.int32)
    batch_ext = jnp.concatenate([batch, jnp.full((_KR,), NSEG, jnp.int32)])

    batch2d = batch.reshape(N_TOK // _TB, 1, _TB)
    partials = _sc_boundaries(batch_ext)
    vn, lp = _sc_gather_vn(partials, session_embedding)
    wqp = jnp.pad(Wq.reshape(HID, 1).astype(jnp.bfloat16), ((0, 0), (0, 127)))
    ae_pad = _tc_alpha_ae(session_embedding, batch2d, vn, W1,
                          (b1 + b2).reshape(1, HID), W2, wqp, bq.reshape(1, 1))
    sg = _sc_segment_sum(batch_ext, lp, ae_pad)
    sh, z = _tc_score(vn, sg, W3[:, :HID], W3[:, HID:],
                      b3.reshape(1, HID), all_item_embedding)
    return sh, z


# speedup vs baseline: 1.7143x; 1.0128x over previous
import functools

import jax
import jax.numpy as jnp
from jax import lax
from jax.experimental import pallas as pl
from jax.experimental.pallas import tpu as pltpu
from jax.experimental.pallas import tpu_sc as plsc

N_TOK = 102400
NSEG = 1024
HID = 256
NVOC = 100000

_NC, _NS, _L = 2, 16, 16
_NW = _NC * _NS
_CHUNK = N_TOK // _NW
_NVEC = _CHUNK // _L
_SEGW = NSEG // _NW
_KR = 256
_JC = _CHUNK // _KR
_ZR = NSEG // _NS

@functools.cache
def _mesh():
    return plsc.VectorSubcoreMesh(core_axis_name="c", subcore_axis_name="s",
                                  num_cores=_NC, num_subcores=_NS)


def _worker_id():
    return lax.axis_index("s") * _NC + lax.axis_index("c")


def _sc_boundaries(batch_ext):
    @functools.partial(
        pl.kernel,
        out_type=jax.ShapeDtypeStruct((_NW * NSEG,), jnp.int32),
        mesh=_mesh(),
        compiler_params=pltpu.CompilerParams(needs_layout_passes=False),
        scratch_types=[
            pltpu.VMEM((_CHUNK + _L,), jnp.int32),
            pltpu.VMEM((NSEG,), jnp.int32),
        ],
    )
    def k(batch_hbm, out_hbm, bvec, last_local):
        wid = _worker_id()
        base = wid * _CHUNK
        pltpu.sync_copy(batch_hbm.at[pl.ds(base, _CHUNK + _L)], bvec)
        zero = jnp.zeros((_L,), jnp.int32)

        def zbody(i, c):
            last_local[pl.ds(i * _L, _L)] = zero
            return c

        lax.fori_loop(0, NSEG // _L, zbody, 0)
        basev = lax.iota(jnp.int32, _L) + (base + 1)

        def body(j, c):
            cur = bvec[pl.ds(j * _L, _L)]
            nxt = bvec[pl.ds(j * _L + 1, _L)]
            vals = basev + j * _L
            plsc.store_scatter(last_local, [cur], vals, mask=cur != nxt)
            return c

        lax.fori_loop(0, _NVEC, body, 0)
        pltpu.sync_copy(last_local, out_hbm.at[pl.ds(wid * NSEG, NSEG)])

    return k(batch_ext)


def _sc_gather_vn(partials, emb):
    @functools.partial(
        pl.kernel,
        out_type=(jax.ShapeDtypeStruct((NSEG, HID), jnp.float32),
                  jax.ShapeDtypeStruct((NSEG,), jnp.int32)),
        mesh=_mesh(),
        compiler_params=pltpu.CompilerParams(needs_layout_passes=False),
        scratch_types=[
            pltpu.VMEM((_NW, _SEGW), jnp.int32),
            pltpu.VMEM((_SEGW,), jnp.int32),
            pltpu.VMEM((_SEGW,), jnp.int32),
            pltpu.VMEM((_SEGW, HID), jnp.float32),
            pltpu.SemaphoreType.DMA,
        ],
    )
    def k(part_hbm, emb_hbm, vn_hbm, lp_hbm, pm, idxv, lpv, rows, sem):
        wid = _worker_id()
        for j in range(_NW):
            pltpu.sync_copy(part_hbm.at[pl.ds(j * NSEG + wid * _SEGW, _SEGW)],
                            pm.at[j])
        acc0 = pm[0, pl.ds(0, _L)]
        acc1 = pm[0, pl.ds(_L, _L)]
        for j in range(1, _NW):
            acc0 = jnp.maximum(acc0, pm[j, pl.ds(0, _L)])
            acc1 = jnp.maximum(acc1, pm[j, pl.ds(_L, _L)])
        lpv[pl.ds(0, _L)] = acc0
        lpv[pl.ds(_L, _L)] = acc1
        idxv[pl.ds(0, _L)] = jnp.maximum(acc0 - 1, 0)
        idxv[pl.ds(_L, _L)] = jnp.maximum(acc1 - 1, 0)
        pltpu.async_copy(emb_hbm.at[idxv], rows, sem).wait()
        pltpu.sync_copy(rows, vn_hbm.at[pl.ds(wid * _SEGW, _SEGW)])
        pltpu.sync_copy(lpv, lp_hbm.at[pl.ds(wid * _SEGW, _SEGW)])

    return k(partials, emb)


_TRASH = _SEGW


def _sc_segment_sum(batch_ext, lp, ae_pad):
    @functools.partial(
        pl.kernel,
        out_type=jax.ShapeDtypeStruct((NSEG, HID), jnp.float32),
        mesh=_mesh(),
        compiler_params=pltpu.CompilerParams(needs_layout_passes=False),
        scratch_types=[
            pltpu.VMEM((NSEG,), jnp.int32),
            pltpu.VMEM((_KR,), jnp.int32),
            pltpu.VMEM((_KR, HID // 2), jnp.uint32),
            pltpu.VMEM((_KR, HID // 2), jnp.uint32),
            pltpu.VMEM((_SEGW + 8, HID), jnp.float32),
            pltpu.SemaphoreType.DMA,
            pltpu.SemaphoreType.DMA,
        ],
    )
    def k(bext_hbm, lp_hbm, ae_hbm, out_hbm, lpv, bv, rows0, rows1, acc, sem0, sem1):
        wid = _worker_id()
        sbase = wid * _SEGW
        pltpu.sync_copy(lp_hbm, lpv)
        zero = jnp.zeros((_L,), jnp.float32)

        def zrow(i, c):
            for q in range(HID // _L):
                acc[i, pl.ds(q * _L, _L)] = zero
            return c

        lax.fori_loop(0, _SEGW + 8, zrow, 0)

        def pbody(v, carry):
            m, s, e = carry
            s = jnp.where(v * _L == sbase, m, s)
            e = jnp.where(v * _L == sbase + _SEGW, m, e)
            m = jnp.maximum(m, jnp.max(lpv[pl.ds(v * _L, _L)]))
            return m, s, e

        m, start, end = lax.fori_loop(0, NSEG // _L, pbody,
                                      (jnp.int32(0), jnp.int32(0), jnp.int32(0)))
        end = jnp.where(sbase + _SEGW == NSEG, m, end)
        astart = (start // 8) * 8
        nch = (end - astart + _KR - 1) // _KR

        hmask = jnp.full((_L,), 0xFFFF0000, jnp.uint32)

        def cvt(u):
            fe = plsc.bitcast(u << 16, jnp.float32)
            fo = plsc.bitcast(u & hmask, jnp.float32)
            return fe, fo

        def accum(rows):
            def vbody(v, c):
                lvec = bv[pl.ds(v * _L, _L)] - sbase
                l0 = lvec[0]
                l15 = lvec[_L - 1]
                r0 = v * _L
                uni = (l0 == l15) & (l0 >= 0) & (l0 < _SEGW)

                @pl.when(uni)
                def _():
                    for j in range(HID // 32):
                        sl = pl.ds(j * _L, _L)
                        es = []
                        os_ = []
                        for t in range(_L):
                            fe, fo = cvt(rows[r0 + t, sl])
                            es.append(fe)
                            os_.append(fo)
                        while len(es) > 1:
                            es = [es[i] + es[i + 1] for i in range(0, len(es), 2)]
                            os_ = [os_[i] + os_[i + 1] for i in range(0, len(os_), 2)]
                        plsc.addupdate(acc.at[l0, pl.ds(j * _L, _L)], es[0])
                        plsc.addupdate(acc.at[l0, pl.ds(128 + j * _L, _L)], os_[0])

                @pl.when(jnp.logical_not(uni))
                def _():
                    for lane in range(_L):
                        lv = lvec[lane]
                        lv = jnp.where((lv >= 0) & (lv < _SEGW), lv, _TRASH)
                        for j in range(HID // 32):
                            fe, fo = cvt(rows[r0 + lane, pl.ds(j * _L, _L)])
                            plsc.addupdate(acc.at[lv, pl.ds(j * _L, _L)], fe)
                            plsc.addupdate(acc.at[lv, pl.ds(128 + j * _L, _L)], fo)
                return c

            lax.fori_loop(0, _KR // _L, vbody, 0)

        @pl.when(nch > 0)
        def _():
            pltpu.async_copy(ae_hbm.at[pl.ds(astart, _KR)], rows0, sem0)

        def body(i, carry):
            c0 = 2 * i
            c1 = c0 + 1
            c2 = c0 + 2
            off0 = astart + c0 * _KR

            @pl.when(c1 < nch)
            def _():
                pltpu.async_copy(ae_hbm.at[pl.ds(off0 + _KR, _KR)], rows1, sem1)

            pltpu.sync_copy(bext_hbm.at[pl.ds(off0, _KR)], bv)
            pltpu.make_async_copy(ae_hbm.at[pl.ds(off0, _KR)], rows0, sem0).wait()
            accum(rows0)

            @pl.when(c2 < nch)
            def _():
                pltpu.async_copy(ae_hbm.at[pl.ds(off0 + 2 * _KR, _KR)], rows0, sem0)

            @pl.when(c1 < nch)
            def _():
                pltpu.sync_copy(bext_hbm.at[pl.ds(off0 + _KR, _KR)], bv)
                pltpu.make_async_copy(ae_hbm.at[pl.ds(off0 + _KR, _KR)], rows1,
                                      sem1).wait()
                accum(rows1)

            return carry

        lax.fori_loop(0, (nch + 1) // 2, body, 0)
        pltpu.sync_copy(acc.at[pl.ds(0, _SEGW)], out_hbm.at[pl.ds(sbase, _SEGW)])

    return k(batch_ext, lp, ae_pad)


_TB = 4096


def _tc_alpha_ae(emb, batch2d, vn, W1, bias, W2, Wqp, bq2):
    def body(emb_ref, b_ref, vn_ref, w1_ref, bias_ref, w2_ref, wqp_ref, bq_ref,
             out_ref, t1_ref):
        @pl.when(pl.program_id(0) == 0)
        def _():
            t1_ref[...] = (
                lax.dot_general(vn_ref[...], w1_ref[...], (((1,), (1,)), ((), ())),
                                preferred_element_type=jnp.float32)
                + bias_ref[...]
            )

        t2 = lax.dot_general(emb_ref[...].astype(jnp.bfloat16),
                             w2_ref[...].astype(jnp.bfloat16),
                             (((1,), (1,)), ((), ())),
                             preferred_element_type=jnp.float32)
        ids = b_ref[0, 0].astype(jnp.int16)
        cols = lax.broadcasted_iota(jnp.int16, (_TB, NSEG), 1)
        oh = (ids.reshape(_TB, 1) == cols).astype(jnp.bfloat16)
        t1x = lax.dot_general(oh, t1_ref[...].astype(jnp.bfloat16),
                              (((1,), (0,)), ((), ())),
                              preferred_element_type=jnp.float32)
        pre = t2 + t1x
        sg = 1.0 / (1.0 + jnp.exp(-pre))
        af = lax.dot_general(sg.astype(jnp.bfloat16), wqp_ref[...],
                             (((1,), (0,)), ((), ())),
                             preferred_element_type=jnp.float32)
        alpha = af[:, 0:1] + bq_ref[...]
        ab = alpha * emb_ref[...]
        lo = lax.bitcast_convert_type(ab[:, :128].astype(jnp.bfloat16),
                                      jnp.uint16).astype(jnp.uint32)
        hi = lax.bitcast_convert_type(ab[:, 128:].astype(jnp.bfloat16),
                                      jnp.uint16).astype(jnp.uint32)
        out_ref[...] = (hi << 16) | lo

    return pl.pallas_call(
        body,
        grid=(N_TOK // _TB,),
        in_specs=[
            pl.BlockSpec((_TB, HID), lambda i: (i, 0)),
            pl.BlockSpec((1, 1, _TB), lambda i: (i, 0, 0)),
            pl.BlockSpec((NSEG, HID), lambda i: (0, 0)),
            pl.BlockSpec((HID, HID), lambda i: (0, 0)),
            pl.BlockSpec((1, HID), lambda i: (0, 0)),
            pl.BlockSpec((HID, HID), lambda i: (0, 0)),
            pl.BlockSpec((HID, 128), lambda i: (0, 0)),
            pl.BlockSpec((1, 1), lambda i: (0, 0)),
        ],
        out_specs=pl.BlockSpec((_TB, HID // 2), lambda i: (i, 0)),
        out_shape=jax.ShapeDtypeStruct((N_TOK + _KR, HID // 2), jnp.uint32),
        scratch_shapes=[pltpu.VMEM((NSEG, HID), jnp.float32)],
    )(emb, batch2d, vn, W1, bias, W2, Wqp, bq2)


_VB = 4096
_NVB = (NVOC + _VB - 1) // _VB


def _tc_score(vn, sg, W3a, W3b, b3r, table):
    def body(vn_ref, sg_ref, w3a_ref, w3b_ref, b3_ref, tbl_ref, sh_ref, z_ref):
        @pl.when(pl.program_id(0) == 0)
        def _():
            sh_ref[...] = (
                lax.dot_general(vn_ref[...], w3a_ref[...], (((1,), (1,)), ((), ())),
                                preferred_element_type=jnp.float32)
                + lax.dot_general(sg_ref[...], w3b_ref[...], (((1,), (1,)), ((), ())),
                                  preferred_element_type=jnp.float32)
                + b3_ref[...]
            )

        z_ref[...] = lax.dot_general(sh_ref[...].astype(jnp.bfloat16),
                                     tbl_ref[...].astype(jnp.bfloat16),
                                     (((1,), (1,)), ((), ())),
                                     preferred_element_type=jnp.float32)

    return pl.pallas_call(
        body,
        grid=(_NVB,),
        in_specs=[
            pl.BlockSpec((NSEG, HID), lambda i: (0, 0)),
            pl.BlockSpec((NSEG, HID), lambda i: (0, 0)),
            pl.BlockSpec((HID, HID), lambda i: (0, 0)),
            pl.BlockSpec((HID, HID), lambda i: (0, 0)),
            pl.BlockSpec((1, HID), lambda i: (0, 0)),
            pl.BlockSpec((_VB, HID), lambda i: (i, 0)),
        ],
        out_specs=[
            pl.BlockSpec((NSEG, HID), lambda i: (0, 0)),
            pl.BlockSpec((NSEG, _VB), lambda i: (0, i)),
        ],
        out_shape=[
            jax.ShapeDtypeStruct((NSEG, HID), jnp.float32),
            jax.ShapeDtypeStruct((NSEG, NVOC), jnp.float32),
        ],
    )(vn, sg, W3a, W3b, b3r, table)


def kernel(session_embedding, batch, all_item_embedding, W1, b1, W2, b2, Wq, bq, W3, b3):
    batch = batch.astype(jnp.int32)
    batch_ext = jnp.concatenate([batch, jnp.full((_KR,), NSEG, jnp.int32)])

    batch2d = batch.reshape(N_TOK // _TB, 1, _TB)
    partials = _sc_boundaries(batch_ext)
    vn, lp = _sc_gather_vn(partials, session_embedding)
    wqp = jnp.pad(Wq.reshape(HID, 1).astype(jnp.bfloat16), ((0, 0), (0, 127)))
    ae_pad = _tc_alpha_ae(session_embedding, batch2d, vn, W1,
                          (b1 + b2).reshape(1, HID), W2, wqp, bq.reshape(1, 1))
    sg = _sc_segment_sum(batch_ext, lp, ae_pad)
    sh, z = _tc_score(vn, sg, W3[:, :HID], W3[:, HID:],
                      b3.reshape(1, HID), all_item_embedding)
    return sh, z
